# baseline (speedup 1.0000x reference)
import jax
import jax.numpy as jnp
from jax.experimental import pallas as pl


def _identity_kernel(x_ref, o_ref):
    o_ref[...] = x_ref[...]


def kernel(q, k, v, edge_index):
    row = edge_index[0].astype(jnp.int32)
    col = edge_index[1].astype(jnp.int32)
    n = q.shape[0]
    q_e = jnp.take(q, row, axis=0)
    k_e = jnp.take(k, col, axis=0)
    attn = jnp.einsum('edh,edh->eh', q_e, k_e)
    row_max = jax.ops.segment_max(attn, row, num_segments=n)
    row_max = jnp.where(jnp.isfinite(row_max), row_max, 0.0)
    attn_exp = jnp.exp(attn - jnp.take(row_max, row, axis=0))
    row_sum = jax.ops.segment_sum(attn_exp, row, num_segments=n)
    row_sum = jnp.where(row_sum == 0.0, 1.0, row_sum)
    attn_sm = attn_exp / jnp.take(row_sum, row, axis=0)
    v_e = jnp.take(v, col, axis=0)
    msgs = attn_sm[:, None, :] * v_e
    out = jax.ops.segment_sum(msgs, row, num_segments=n)
    out2 = out.reshape(n, 256)
    res = pl.pallas_call(
        _identity_kernel,
        grid=(25,),
        in_specs=[pl.BlockSpec((400, 256), lambda i: (i, 0))],
        out_specs=pl.BlockSpec((400, 256), lambda i: (i, 0)),
        out_shape=jax.ShapeDtypeStruct((n, 256), out.dtype),
    )(out2)
    return res.reshape(n, 32, 8)


# speedup vs baseline: 3.7459x; 3.7459x over previous
import functools

import jax
import jax.numpy as jnp
from jax import lax
from jax.experimental import pallas as pl
from jax.experimental.pallas import tpu as pltpu
from jax.experimental.pallas import tpu_sc as plsc

N = 10000
E = 160000
NHEAD = 8
DH = 32
HPC = 4
DC = HPC * DH
NPAD = 10024
DUMP = 10016
NPASS = 4
HALF = 2560
NSHP = 3072
RPT = NSHP // 16
EPAD = 163840
G = 32
BPT = EPAD // 16 // G


def _sc_body(qcat, kcat, vcat, rows3_hbm, rows2_hbm, cols2_hbm, out_hbm,
             rows_b, rows_g, cols_g, qb, kb, vb, rsm,
             out_sh, rs_sh, sem0, sem1, sem2):
    c = lax.axis_index("c")
    s = lax.axis_index("s")
    zf = jnp.zeros((16,), jnp.float32)
    ix = lax.iota(jnp.int32, 16)
    lane4 = jnp.bitwise_and(ix, 3)
    chunk0 = s * (EPAD // 16)
    row0 = s * RPT

    def _zrow(r, carry):
        for j in range(DC // 16):
            vb[r, pl.ds(j * 16, 16)] = zf
            rsm[r, pl.ds(j * 16, 16)] = zf
        return carry

    def _zcopy(bb, carry):
        base = row0 + bb * G
        pltpu.sync_copy(vb, out_sh.at[pl.ds(base, G)])
        pltpu.sync_copy(rsm, rs_sh.at[pl.ds(base, G)])
        return carry

    def _edge(e, carry):
        ws = []
        for h in range(HPC):
            qa = qb[e, pl.ds(h * 32, 16)]
            qc2 = qb[e, pl.ds(h * 32 + 16, 16)]
            ka = kb[e, pl.ds(h * 32, 16)]
            kc2 = kb[e, pl.ds(h * 32 + 16, 16)]
            p = qa * ka + qc2 * kc2
            ah = jnp.sum(p)
            wh = jnp.exp(jnp.full((16,), ah, jnp.float32))
            vb[e, pl.ds(h * 32, 16)] = wh * vb[e, pl.ds(h * 32, 16)]
            vb[e, pl.ds(h * 32 + 16, 16)] = wh * vb[e, pl.ds(h * 32 + 16, 16)]
            ws.append(wh)
        rs = jnp.where(lane4 == 0, ws[0],
                       jnp.where(lane4 == 1, ws[1],
                                 jnp.where(lane4 == 2, ws[2], ws[3])))
        for j in range(DC // 16):
            rsm[e, pl.ds(j * 16, 16)] = rs
        return carry

    def _nrow(r, carry):
        rsv = rsm[r, pl.ds(0, 16)]
        rsv = jnp.where(rsv == 0.0, 1.0, rsv)
        rec = 1.0 / rsv
        for h in range(HPC):
            rh = jnp.max(jnp.where(lane4 == h, rec, 0.0))
            rhv = jnp.full((16,), rh, jnp.float32)
            qb[r, pl.ds(h * 32, 16)] = qb[r, pl.ds(h * 32, 16)] * rhv
            qb[r, pl.ds(h * 32 + 16, 16)] = qb[r, pl.ds(h * 32 + 16, 16)] * rhv
        return carry

    for p in range(NPASS):
        lax.fori_loop(0, G, _zrow, 0)
        lax.fori_loop(0, RPT // G, _zcopy, 0)
        plsc.subcore_barrier()

        def _batch(b, carry, p=p):
            base = chunk0 + b * G
            pltpu.sync_copy(rows3_hbm.at[pl.ds(p * EPAD + base, G)], rows_b)
            pltpu.sync_copy(rows2_hbm.at[pl.ds(c * EPAD + base, G)], rows_g)
            pltpu.sync_copy(cols2_hbm.at[pl.ds(c * EPAD + base, G)], cols_g)
            cq = pltpu.async_copy(qcat.at[rows_g], qb, sem0)
            ck = pltpu.async_copy(kcat.at[cols_g], kb, sem1)
            cv = pltpu.async_copy(vcat.at[cols_g], vb, sem2)
            cq.wait()
            ck.wait()
            cv.wait()
            lax.fori_loop(0, G, _edge, 0)
            pltpu.sync_copy(vb, out_sh.at[rows_b], add=True)
            pltpu.sync_copy(rsm, rs_sh.at[rows_b], add=True)
            return carry

        lax.fori_loop(0, BPT, _batch, 0)
        plsc.subcore_barrier()

        def _nchunk(nb, carry, p=p):
            base = row0 + nb * G
            pltpu.sync_copy(out_sh.at[pl.ds(base, G)], qb)
            pltpu.sync_copy(rs_sh.at[pl.ds(base, G)], rsm)
            lax.fori_loop(0, G, _nrow, 0)
            pltpu.sync_copy(
                qb, out_hbm.at[pl.ds((NPASS * c + p) * NSHP + base, G)])
            return carry

        lax.fori_loop(0, RPT // G, _nchunk, 0)
        plsc.subcore_barrier()


@jax.jit
def kernel(q, k, v, edge_index):
    row = edge_index[0].astype(jnp.int32)
    col = edge_index[1].astype(jnp.int32)
    epad = EPAD - E
    rows_pad = jnp.concatenate([row, jnp.full((epad,), DUMP, jnp.int32)])
    cols_pad = jnp.concatenate([col, jnp.zeros((epad,), jnp.int32)])
    rows2 = jnp.concatenate([rows_pad, rows_pad + NPAD])
    cols2 = jnp.concatenate([cols_pad, cols_pad + NPAD])
    spread = HALF + jnp.bitwise_and(jnp.arange(EPAD, dtype=jnp.int32), 255)
    halves = []
    for p in range(NPASS):
        in_half = jnp.logical_and(rows_pad >= p * HALF,
                                  rows_pad < (p + 1) * HALF)
        halves.append(jnp.where(in_half, rows_pad - p * HALF, spread))
    rows3 = jnp.concatenate(halves)

    def prep(x):
        xt = jnp.transpose(x, (0, 2, 1))
        parts = jnp.stack([xt[:, :HPC].reshape(N, DC),
                           xt[:, HPC:].reshape(N, DC)])
        return jnp.pad(parts, ((0, 0), (0, NPAD - N), (0, 0))).reshape(
            2 * NPAD, DC)

    qcat, kcat, vcat = prep(q), prep(k), prep(v)

    mesh = plsc.VectorSubcoreMesh(core_axis_name="c", subcore_axis_name="s",
                                  num_cores=2, num_subcores=16)
    scf = functools.partial(
        pl.kernel,
        out_type=jax.ShapeDtypeStruct((2 * NPASS * NSHP, DC), jnp.float32),
        mesh=mesh,
        compiler_params=pltpu.CompilerParams(needs_layout_passes=False),
        scratch_types=[
            pltpu.VMEM((G,), jnp.int32),
            pltpu.VMEM((G,), jnp.int32),
            pltpu.VMEM((G,), jnp.int32),
            pltpu.VMEM((G, DC), jnp.float32),
            pltpu.VMEM((G, DC), jnp.float32),
            pltpu.VMEM((G, DC), jnp.float32),
            pltpu.VMEM((G, DC), jnp.float32),
            pltpu.VMEM_SHARED((NSHP, DC), jnp.float32),
            pltpu.VMEM_SHARED((NSHP, DC), jnp.float32),
            pltpu.SemaphoreType.DMA,
            pltpu.SemaphoreType.DMA,
            pltpu.SemaphoreType.DMA,
        ],
    )
    out2 = scf(_sc_body)(qcat, kcat, vcat, rows3, rows2, cols2)
    o = out2.reshape(2, NPASS, NSHP, DC)[:, :, :HALF]
    o = o.reshape(2, NPASS * HALF, HPC, DH)[:, :N]
    return o.transpose(1, 3, 0, 2).reshape(N, DH, NHEAD)


# speedup vs baseline: 4.9534x; 1.3224x over previous
import functools

import jax
import jax.numpy as jnp
from jax import lax
from jax.experimental import pallas as pl
from jax.experimental.pallas import tpu as pltpu
from jax.experimental.pallas import tpu_sc as plsc

N = 10000
E = 160000
NHEAD = 8
DH = 32
HPC = 4
DC = HPC * DH
NPAD = 10024
DUMP = 10016
NPASS = 4
HALF = 2560
NSHP = 3072
RPT = NSHP // 16
EPAD = 163840
G = 64
BPT = EPAD // 16 // G


def _sc_body(qcat, kcat, vcat, rows3_hbm, rows2_hbm, cols2_hbm, out_hbm,
             rows_b, rows_g, cols_g, qb, kb, vb, rsm,
             out_sh, rs_sh, sem0, sem1, sem2):
    c = lax.axis_index("c")
    s = lax.axis_index("s")
    zf = jnp.zeros((16,), jnp.float32)
    ix = lax.iota(jnp.int32, 16)
    lane4 = jnp.bitwise_and(ix, 3)
    chunk0 = s * (EPAD // 16)
    row0 = s * RPT

    def _zrow(r, carry):
        for j in range(DC // 16):
            vb[r, pl.ds(j * 16, 16)] = zf
            rsm[r, pl.ds(j * 16, 16)] = zf
        return carry

    def _zcopy(bb, carry):
        base = row0 + bb * G
        pltpu.sync_copy(vb, out_sh.at[pl.ds(base, G)])
        pltpu.sync_copy(rsm, rs_sh.at[pl.ds(base, G)])
        return carry

    def _edge(e, carry):
        ws = []
        for h in range(HPC):
            qa = qb[e, pl.ds(h * 32, 16)]
            qc2 = qb[e, pl.ds(h * 32 + 16, 16)]
            ka = kb[e, pl.ds(h * 32, 16)]
            kc2 = kb[e, pl.ds(h * 32 + 16, 16)]
            p = qa * ka + qc2 * kc2
            ah = jnp.sum(p)
            wh = jnp.exp(jnp.full((16,), ah, jnp.float32))
            vb[e, pl.ds(h * 32, 16)] = wh * vb[e, pl.ds(h * 32, 16)]
            vb[e, pl.ds(h * 32 + 16, 16)] = wh * vb[e, pl.ds(h * 32 + 16, 16)]
            ws.append(wh)
        rs = jnp.where(lane4 == 0, ws[0],
                       jnp.where(lane4 == 1, ws[1],
                                 jnp.where(lane4 == 2, ws[2], ws[3])))
        for j in range(DC // 16):
            rsm[e, pl.ds(j * 16, 16)] = rs
        return carry

    def _nrow(r, carry):
        rsv = rsm[r, pl.ds(0, 16)]
        rsv = jnp.where(rsv == 0.0, 1.0, rsv)
        rec = 1.0 / rsv
        for h in range(HPC):
            rh = jnp.max(jnp.where(lane4 == h, rec, 0.0))
            rhv = jnp.full((16,), rh, jnp.float32)
            qb[r, pl.ds(h * 32, 16)] = qb[r, pl.ds(h * 32, 16)] * rhv
            qb[r, pl.ds(h * 32 + 16, 16)] = qb[r, pl.ds(h * 32 + 16, 16)] * rhv
        return carry

    for p in range(NPASS):
        lax.fori_loop(0, G, _zrow, 0)
        lax.fori_loop(0, RPT // G, _zcopy, 0)
        plsc.subcore_barrier()

        def _batch(b, carry, p=p):
            base = chunk0 + b * G
            pltpu.sync_copy(rows3_hbm.at[pl.ds(p * EPAD + base, G)], rows_b)
            pltpu.sync_copy(rows2_hbm.at[pl.ds(c * EPAD + base, G)], rows_g)
            pltpu.sync_copy(cols2_hbm.at[pl.ds(c * EPAD + base, G)], cols_g)
            cq = pltpu.async_copy(qcat.at[rows_g], qb, sem0)
            ck = pltpu.async_copy(kcat.at[cols_g], kb, sem1)
            cv = pltpu.async_copy(vcat.at[cols_g], vb, sem2)
            cq.wait()
            ck.wait()
            cv.wait()
            lax.fori_loop(0, G, _edge, 0)
            pltpu.sync_copy(vb, out_sh.at[rows_b], add=True)
            pltpu.sync_copy(rsm, rs_sh.at[rows_b], add=True)
            return carry

        lax.fori_loop(0, BPT, _batch, 0)
        plsc.subcore_barrier()

        def _nchunk(nb, carry, p=p):
            base = row0 + nb * G
            pltpu.sync_copy(out_sh.at[pl.ds(base, G)], qb)
            pltpu.sync_copy(rs_sh.at[pl.ds(base, G)], rsm)
            lax.fori_loop(0, G, _nrow, 0)
            pltpu.sync_copy(
                qb, out_hbm.at[pl.ds((NPASS * c + p) * NSHP + base, G)])
            return carry

        lax.fori_loop(0, RPT // G, _nchunk, 0)
        plsc.subcore_barrier()


@jax.jit
def kernel(q, k, v, edge_index):
    row = edge_index[0].astype(jnp.int32)
    col = edge_index[1].astype(jnp.int32)
    epad = EPAD - E
    rows_pad = jnp.concatenate([row, jnp.full((epad,), DUMP, jnp.int32)])
    cols_pad = jnp.concatenate([col, jnp.zeros((epad,), jnp.int32)])
    rows2 = jnp.concatenate([rows_pad, rows_pad + NPAD])
    cols2 = jnp.concatenate([cols_pad, cols_pad + NPAD])
    spread = HALF + jnp.bitwise_and(jnp.arange(EPAD, dtype=jnp.int32), 255)
    halves = []
    for p in range(NPASS):
        in_half = jnp.logical_and(rows_pad >= p * HALF,
                                  rows_pad < (p + 1) * HALF)
        halves.append(jnp.where(in_half, rows_pad - p * HALF, spread))
    rows3 = jnp.concatenate(halves)

    def prep(x):
        xt = jnp.transpose(x, (0, 2, 1))
        parts = jnp.stack([xt[:, :HPC].reshape(N, DC),
                           xt[:, HPC:].reshape(N, DC)])
        return jnp.pad(parts, ((0, 0), (0, NPAD - N), (0, 0))).reshape(
            2 * NPAD, DC)

    qcat, kcat, vcat = prep(q), prep(k), prep(v)

    mesh = plsc.VectorSubcoreMesh(core_axis_name="c", subcore_axis_name="s",
                                  num_cores=2, num_subcores=16)
    scf = functools.partial(
        pl.kernel,
        out_type=jax.ShapeDtypeStruct((2 * NPASS * NSHP, DC), jnp.float32),
        mesh=mesh,
        compiler_params=pltpu.CompilerParams(needs_layout_passes=False),
        scratch_types=[
            pltpu.VMEM((G,), jnp.int32),
            pltpu.VMEM((G,), jnp.int32),
            pltpu.VMEM((G,), jnp.int32),
            pltpu.VMEM((G, DC), jnp.float32),
            pltpu.VMEM((G, DC), jnp.float32),
            pltpu.VMEM((G, DC), jnp.float32),
            pltpu.VMEM((G, DC), jnp.float32),
            pltpu.VMEM_SHARED((NSHP, DC), jnp.float32),
            pltpu.VMEM_SHARED((NSHP, DC), jnp.float32),
            pltpu.SemaphoreType.DMA,
            pltpu.SemaphoreType.DMA,
            pltpu.SemaphoreType.DMA,
        ],
    )
    out2 = scf(_sc_body)(qcat, kcat, vcat, rows3, rows2, cols2)
    o = out2.reshape(2, NPASS, NSHP, DC)[:, :, :HALF]
    o = o.reshape(2, NPASS * HALF, HPC, DH)[:, :N]
    return o.transpose(1, 3, 0, 2).reshape(N, DH, NHEAD)


# speedup vs baseline: 5.4553x; 1.1013x over previous
import functools

import jax
import jax.numpy as jnp
from jax import lax
from jax.experimental import pallas as pl
from jax.experimental.pallas import tpu as pltpu
from jax.experimental.pallas import tpu_sc as plsc

N = 10000
E = 160000
NHEAD = 8
DH = 32
HPC = 4
DC = HPC * DH
NPAD = 10024
DUMP = 10016
NPASS = 4
HALF = 2560
NSHP = 3072
RPT = NSHP // 16
EPAD = 163840
G = 64
BPT = EPAD // 16 // G


def _sc_body(qcat, kcat, vcat, rows3_hbm, rows2_hbm, cols2_hbm, out_hbm,
             rows_b, rows_g, cols_g, qb, kb, vb, rsm,
             out_sh, rs_sh, sem0, sem1, sem2):
    c = lax.axis_index("c")
    s = lax.axis_index("s")
    zf = jnp.zeros((16,), jnp.float32)
    ix = lax.iota(jnp.int32, 16)
    lane4 = jnp.bitwise_and(ix, 3)
    chunk0 = s * (EPAD // 16)
    row0 = s * RPT

    def _zrow(r, carry):
        for j in range(DC // 16):
            vb[r, pl.ds(j * 16, 16)] = zf
            rsm[r, pl.ds(j * 16, 16)] = zf
        return carry

    def _zcopy(bb, carry):
        base = row0 + bb * G
        pltpu.sync_copy(vb, out_sh.at[pl.ds(base, G)])
        pltpu.sync_copy(rsm, rs_sh.at[pl.ds(base, G)])
        return carry

    def _edge(eh, carry):
        for u in range(2):
            e = eh * 2 + u
            ws = []
            for h in range(HPC):
                qa = qb[e, pl.ds(h * 32, 16)]
                qc2 = qb[e, pl.ds(h * 32 + 16, 16)]
                ka = kb[e, pl.ds(h * 32, 16)]
                kc2 = kb[e, pl.ds(h * 32 + 16, 16)]
                p = qa * ka + qc2 * kc2
                ah = jnp.sum(p)
                wh = jnp.exp(jnp.full((16,), ah, jnp.float32))
                vb[e, pl.ds(h * 32, 16)] = wh * vb[e, pl.ds(h * 32, 16)]
                vb[e, pl.ds(h * 32 + 16, 16)] = (
                    wh * vb[e, pl.ds(h * 32 + 16, 16)])
                ws.append(wh)
            rs = jnp.where(lane4 == 0, ws[0],
                           jnp.where(lane4 == 1, ws[1],
                                     jnp.where(lane4 == 2, ws[2], ws[3])))
            for j in range(DC // 16):
                rsm[e, pl.ds(j * 16, 16)] = rs
        return carry

    def _nrow(r, carry):
        rsv = rsm[r, pl.ds(0, 16)]
        rsv = jnp.where(rsv == 0.0, 1.0, rsv)
        rec = 1.0 / rsv
        for h in range(HPC):
            rh = jnp.max(jnp.where(lane4 == h, rec, 0.0))
            rhv = jnp.full((16,), rh, jnp.float32)
            qb[r, pl.ds(h * 32, 16)] = qb[r, pl.ds(h * 32, 16)] * rhv
            qb[r, pl.ds(h * 32 + 16, 16)] = qb[r, pl.ds(h * 32 + 16, 16)] * rhv
        return carry

    for p in range(NPASS):
        lax.fori_loop(0, G, _zrow, 0)
        lax.fori_loop(0, RPT // G, _zcopy, 0)
        plsc.subcore_barrier()

        def _batch(b, carry, p=p):
            base = chunk0 + b * G
            c1 = pltpu.async_copy(
                rows3_hbm.at[pl.ds(p * EPAD + base, G)], rows_b, sem0)
            c2 = pltpu.async_copy(
                rows2_hbm.at[pl.ds(c * EPAD + base, G)], rows_g, sem1)
            c3 = pltpu.async_copy(
                cols2_hbm.at[pl.ds(c * EPAD + base, G)], cols_g, sem2)
            c1.wait()
            c2.wait()
            c3.wait()
            cq = pltpu.async_copy(qcat.at[rows_g], qb, sem0)
            ck = pltpu.async_copy(kcat.at[cols_g], kb, sem1)
            cv = pltpu.async_copy(vcat.at[cols_g], vb, sem2)
            cq.wait()
            ck.wait()
            cv.wait()
            lax.fori_loop(0, G // 2, _edge, 0)
            s1 = pltpu.async_copy(vb, out_sh.at[rows_b], sem0, add=True)
            s2 = pltpu.async_copy(rsm, rs_sh.at[rows_b], sem1, add=True)
            s1.wait()
            s2.wait()
            return carry

        lax.fori_loop(0, BPT, _batch, 0)
        plsc.subcore_barrier()

        def _nchunk(nb, carry, p=p):
            base = row0 + nb * G
            pltpu.sync_copy(out_sh.at[pl.ds(base, G)], qb)
            pltpu.sync_copy(rs_sh.at[pl.ds(base, G)], rsm)
            lax.fori_loop(0, G, _nrow, 0)
            pltpu.sync_copy(
                qb, out_hbm.at[pl.ds((NPASS * c + p) * NSHP + base, G)])
            return carry

        lax.fori_loop(0, RPT // G, _nchunk, 0)
        plsc.subcore_barrier()


@jax.jit
def kernel(q, k, v, edge_index):
    row = edge_index[0].astype(jnp.int32)
    col = edge_index[1].astype(jnp.int32)
    epad = EPAD - E
    rows_pad = jnp.concatenate([row, jnp.full((epad,), DUMP, jnp.int32)])
    cols_pad = jnp.concatenate([col, jnp.zeros((epad,), jnp.int32)])
    rows2 = jnp.concatenate([rows_pad, rows_pad + NPAD])
    cols2 = jnp.concatenate([cols_pad, cols_pad + NPAD])
    spread = HALF + jnp.bitwise_and(jnp.arange(EPAD, dtype=jnp.int32), 255)
    halves = []
    for p in range(NPASS):
        in_half = jnp.logical_and(rows_pad >= p * HALF,
                                  rows_pad < (p + 1) * HALF)
        halves.append(jnp.where(in_half, rows_pad - p * HALF, spread))
    rows3 = jnp.concatenate(halves)

    def prep(x):
        xt = jnp.transpose(x, (0, 2, 1))
        parts = jnp.stack([xt[:, :HPC].reshape(N, DC),
                           xt[:, HPC:].reshape(N, DC)])
        return jnp.pad(parts, ((0, 0), (0, NPAD - N), (0, 0))).reshape(
            2 * NPAD, DC)

    qcat, kcat, vcat = prep(q), prep(k), prep(v)

    mesh = plsc.VectorSubcoreMesh(core_axis_name="c", subcore_axis_name="s",
                                  num_cores=2, num_subcores=16)
    scf = functools.partial(
        pl.kernel,
        out_type=jax.ShapeDtypeStruct((2 * NPASS * NSHP, DC), jnp.float32),
        mesh=mesh,
        compiler_params=pltpu.CompilerParams(needs_layout_passes=False),
        scratch_types=[
            pltpu.VMEM((G,), jnp.int32),
            pltpu.VMEM((G,), jnp.int32),
            pltpu.VMEM((G,), jnp.int32),
            pltpu.VMEM((G, DC), jnp.float32),
            pltpu.VMEM((G, DC), jnp.float32),
            pltpu.VMEM((G, DC), jnp.float32),
            pltpu.VMEM((G, DC), jnp.float32),
            pltpu.VMEM_SHARED((NSHP, DC), jnp.float32),
            pltpu.VMEM_SHARED((NSHP, DC), jnp.float32),
            pltpu.SemaphoreType.DMA,
            pltpu.SemaphoreType.DMA,
            pltpu.SemaphoreType.DMA,
        ],
    )
    out2 = scf(_sc_body)(qcat, kcat, vcat, rows3, rows2, cols2)
    o = out2.reshape(2, NPASS, NSHP, DC)[:, :, :HALF]
    o = o.reshape(2, NPASS * HALF, HPC, DH)[:, :N]
    return o.transpose(1, 3, 0, 2).reshape(N, DH, NHEAD)
